# baseline (speedup 1.0000x reference)
import functools

import jax
import jax.numpy as jnp
from jax import lax
from jax.experimental import pallas as pl
from jax.experimental.pallas import tpu as pltpu
from jax.experimental.pallas import tpu_sc as plsc

L = 2048
D = 2048
H = 16
K = 512
BQ = 1024
NQ = L // BQ
NSPLIT = 2
HS = H // NSPLIT
GRID = (HS, NQ)


def _lane_cumsum(x):
    col = lax.broadcasted_iota(jnp.int32, x.shape, 1)
    s = 1
    while s < x.shape[1]:
        rolled = pltpu.roll(x, s, 1)
        x = x + jnp.where(col >= s, rolled, 0.0)
        s *= 2
    return x


def _importance_topk_kernel(scores0_ref, scores1_ref, amask_ref, idx_ref,
                            pmask_ref, acc_ref):
    h = pl.program_id(0)
    qi = pl.program_id(1)
    step = h * NQ + qi

    @pl.when(step == 0)
    def _():
        acc_ref[...] = jnp.zeros_like(acc_ref)

    part = (jnp.sum(scores0_ref[0], axis=0, keepdims=True)
            + jnp.sum(scores1_ref[0], axis=0, keepdims=True))
    acc_ref[...] += part

    @pl.when(step == HS * NQ - 1)
    def _():
        x = acc_ref[...]
        bits = lax.bitcast_convert_type(x, jnp.int32)
        skey = jnp.where(bits >= 0, bits, bits ^ jnp.int32(0x7FFFFFFF))

        def bit_step(i, p):
            trial = p + (jnp.int32(1) << (jnp.int32(31) - i))
            c = jnp.sum((skey >= trial).astype(jnp.int32))
            return jnp.where(c >= K, trial, p)

        t = lax.fori_loop(0, 32, bit_step, jnp.int32(-2147483648))

        gt = skey > t
        eq = skey == t
        need = K - jnp.sum(gt.astype(jnp.int32))
        cum_eq = _lane_cumsum(eq.astype(jnp.float32))
        sel = gt | (eq & (cum_eq <= need.astype(jnp.float32)))

        rank = (_lane_cumsum(sel.astype(jnp.float32)) - 1.0).astype(jnp.int32)
        j = lax.broadcasted_iota(jnp.int32, (K, L), 0)
        onehot = jnp.where((rank == j) & sel, 1.0, 0.0)
        lpos = lax.broadcasted_iota(jnp.int32, (K, L), 1).astype(jnp.float32)
        idx_ref[...] = jnp.sum(onehot * lpos, axis=1, keepdims=True).astype(
            jnp.int32).reshape(1, K)
        pmask_ref[...] = jnp.sum(onehot * amask_ref[...], axis=1,
                                 keepdims=True).reshape(1, K)


_importance_topk = pl.pallas_call(
    _importance_topk_kernel,
    grid=GRID,
    in_specs=[
        pl.BlockSpec((1, BQ, L), lambda h, qi: (h, qi, 0)),
        pl.BlockSpec((1, BQ, L), lambda h, qi: (h, qi, 0)),
        pl.BlockSpec((1, L), lambda h, qi: (0, 0)),
    ],
    out_specs=[
        pl.BlockSpec((1, K), lambda h, qi: (0, 0)),
        pl.BlockSpec((1, K), lambda h, qi: (0, 0)),
    ],
    out_shape=[
        jax.ShapeDtypeStruct((1, K), jnp.int32),
        jax.ShapeDtypeStruct((1, K), jnp.float32),
    ],
    scratch_shapes=[pltpu.VMEM((1, L), jnp.float32)],
)


@functools.cache
def _make_sc_gather():
    info = plsc.get_sparse_core_info()
    nc, ns = info.num_cores, info.num_subcores
    nw = nc * ns
    rows_per_w = (K + 1) // nw

    @functools.partial(
        pl.kernel,
        out_type=jax.ShapeDtypeStruct((K + 1, D), jnp.float32),
        mesh=plsc.VectorSubcoreMesh(core_axis_name="c", subcore_axis_name="s"),
        scratch_types=[
            pltpu.VMEM((rows_per_w,), jnp.int32),
            pltpu.VMEM((rows_per_w, D), jnp.float32),
            pltpu.VMEM((1,), jnp.int32),
            pltpu.VMEM((1, D), jnp.float32),
            pltpu.SemaphoreType.DMA,
        ],
    )
    def _sc_gather(table_hbm, idx_hbm, out_hbm, idx_v, rows_v, idx_e, row_e,
                   sem):
        wid = lax.axis_index("s") * nc + lax.axis_index("c")
        base = wid * rows_per_w
        pltpu.sync_copy(idx_hbm.at[pl.ds(base, rows_per_w)], idx_v)
        pltpu.async_copy(table_hbm.at[idx_v], rows_v, sem).wait()
        pltpu.sync_copy(rows_v, out_hbm.at[pl.ds(base, rows_per_w)])

        @pl.when(wid == nw - 1)
        def _():
            last = nw * rows_per_w
            pltpu.sync_copy(idx_hbm.at[pl.ds(last, 1)], idx_e)
            pltpu.async_copy(table_hbm.at[idx_e], row_e, sem).wait()
            pltpu.sync_copy(row_e, out_hbm.at[pl.ds(last, 1)])

    return _sc_gather


def kernel(hidden_states, attention_mask, self_attention_scores,
           Wq, bq, Wk, bk, Wv, bv, Wo, bo):
    b = hidden_states.shape[0]
    scores = self_attention_scores.reshape(H, L, L)
    amask_row = attention_mask.reshape(1, L)

    idx, pmask = _importance_topk(scores[:HS], scores[HS:], amask_row)

    table = hidden_states.reshape(L, D)
    idx_full = jnp.concatenate(
        [jnp.zeros((1,), jnp.int32), idx.reshape(K)])
    final = _make_sc_gather()(table, idx_full)

    final_token = final.reshape(b, K + 1, D)
    final_attention_mask = jnp.concatenate(
        [jnp.zeros((1, 1), attention_mask.dtype), pmask], axis=1
    ).reshape(b, 1, 1, K + 1)
    return (final_token, final_attention_mask)


# speedup vs baseline: 2.4202x; 2.4202x over previous
import functools

import jax
import jax.numpy as jnp
from jax import lax
from jax.experimental import pallas as pl
from jax.experimental.pallas import tpu as pltpu
from jax.experimental.pallas import tpu_sc as plsc

L = 2048
D = 2048
H = 16
K = 512
BQ = 1024
NQ = L // BQ
NSPLIT = 2
HS = H // NSPLIT
GRID = (HS, NQ)


def _lane_cumsum(x):
    col = lax.broadcasted_iota(jnp.int32, x.shape, 1)
    s = 1
    while s < x.shape[1]:
        rolled = pltpu.roll(x, s, 1)
        x = x + jnp.where(col >= s, rolled, 0.0)
        s *= 2
    return x


def _importance_topk_kernel(scores0_ref, scores1_ref, amask_ref, idx_ref,
                            pmask_ref, acc_ref):
    h = pl.program_id(0)
    qi = pl.program_id(1)
    step = h * NQ + qi

    @pl.when(step == 0)
    def _():
        acc_ref[...] = jnp.zeros_like(acc_ref)

    part = (jnp.sum(scores0_ref[0], axis=0, keepdims=True)
            + jnp.sum(scores1_ref[0], axis=0, keepdims=True))
    acc_ref[...] += part

    @pl.when(step == HS * NQ - 1)
    def _():
        x = acc_ref[...]
        bits = lax.bitcast_convert_type(x, jnp.int32)
        skey = jnp.where(bits >= 0, bits, bits ^ jnp.int32(0x7FFFFFFF))

        def bit_step(i, p):
            trial = p + (jnp.int32(1) << (jnp.int32(31) - i))
            c = jnp.sum((skey >= trial).astype(jnp.int32))
            return jnp.where(c >= K, trial, p)

        t = lax.fori_loop(0, 32, bit_step, jnp.int32(-2147483648))

        gt = skey > t
        eq = skey == t
        need = K - jnp.sum(gt.astype(jnp.int32))
        cum_eq = _lane_cumsum(eq.astype(jnp.float32))
        sel = gt | (eq & (cum_eq <= need.astype(jnp.float32)))

        rank = (_lane_cumsum(sel.astype(jnp.float32)) - 1.0).astype(jnp.int32)
        j = lax.broadcasted_iota(jnp.int32, (K, L), 0)
        onehot = jnp.where((rank == j) & sel, 1.0, 0.0)
        lpos = lax.broadcasted_iota(jnp.int32, (K, L), 1).astype(jnp.float32)
        idx_ref[...] = jnp.sum(onehot * lpos, axis=1, keepdims=True).astype(
            jnp.int32).reshape(1, K)
        pmask_ref[...] = jnp.sum(onehot * amask_ref[...], axis=1,
                                 keepdims=True).reshape(1, K)


_importance_topk = pl.pallas_call(
    _importance_topk_kernel,
    grid=GRID,
    in_specs=[
        pl.BlockSpec((1, BQ, L), lambda h, qi: (h, qi, 0)),
        pl.BlockSpec((1, BQ, L), lambda h, qi: (h + HS, qi, 0)),
        pl.BlockSpec((1, L), lambda h, qi: (0, 0)),
    ],
    out_specs=[
        pl.BlockSpec((1, K), lambda h, qi: (0, 0)),
        pl.BlockSpec((1, K), lambda h, qi: (0, 0)),
    ],
    out_shape=[
        jax.ShapeDtypeStruct((1, K), jnp.int32),
        jax.ShapeDtypeStruct((1, K), jnp.float32),
    ],
    scratch_shapes=[pltpu.VMEM((1, L), jnp.float32)],
)


@functools.cache
def _make_sc_gather():
    info = plsc.get_sparse_core_info()
    nc, ns = info.num_cores, info.num_subcores
    nw = nc * ns
    rows_per_w = (K + 1) // nw

    @functools.partial(
        pl.kernel,
        out_type=jax.ShapeDtypeStruct((K + 1, D), jnp.float32),
        mesh=plsc.VectorSubcoreMesh(core_axis_name="c", subcore_axis_name="s"),
        scratch_types=[
            pltpu.VMEM((rows_per_w,), jnp.int32),
            pltpu.VMEM((rows_per_w, D), jnp.float32),
            pltpu.VMEM((1,), jnp.int32),
            pltpu.VMEM((1, D), jnp.float32),
            pltpu.SemaphoreType.DMA,
        ],
    )
    def _sc_gather(table_hbm, idx_hbm, out_hbm, idx_v, rows_v, idx_e, row_e,
                   sem):
        wid = lax.axis_index("s") * nc + lax.axis_index("c")
        base = wid * rows_per_w
        pltpu.sync_copy(idx_hbm.at[pl.ds(base, rows_per_w)], idx_v)
        pltpu.async_copy(table_hbm.at[idx_v], rows_v, sem).wait()
        pltpu.sync_copy(rows_v, out_hbm.at[pl.ds(base, rows_per_w)])

        @pl.when(wid == nw - 1)
        def _():
            last = nw * rows_per_w
            pltpu.sync_copy(idx_hbm.at[pl.ds(last, 1)], idx_e)
            pltpu.async_copy(table_hbm.at[idx_e], row_e, sem).wait()
            pltpu.sync_copy(row_e, out_hbm.at[pl.ds(last, 1)])

    return _sc_gather


def kernel(hidden_states, attention_mask, self_attention_scores,
           Wq, bq, Wk, bk, Wv, bv, Wo, bo):
    b = hidden_states.shape[0]
    scores = self_attention_scores.reshape(H, L, L)
    amask_row = attention_mask.reshape(1, L)

    idx, pmask = _importance_topk(scores, scores, amask_row)

    table = hidden_states.reshape(L, D)
    idx_full = jnp.concatenate(
        [jnp.zeros((1,), jnp.int32), idx.reshape(K)])
    final = _make_sc_gather()(table, idx_full)

    final_token = final.reshape(b, K + 1, D)
    final_attention_mask = jnp.concatenate(
        [jnp.zeros((1, 1), attention_mask.dtype), pmask], axis=1
    ).reshape(b, 1, 1, K + 1)
    return (final_token, final_attention_mask)


# speedup vs baseline: 2.4254x; 1.0022x over previous
import functools

import jax
import jax.numpy as jnp
from jax import lax
from jax.experimental import pallas as pl
from jax.experimental.pallas import tpu as pltpu
from jax.experimental.pallas import tpu_sc as plsc

L = 2048
D = 2048
H = 16
K = 512
BQ = 1024
NQ = L // BQ
GRID = (H, NQ)


def _lane_cumsum(x):
    col = lax.broadcasted_iota(jnp.int32, x.shape, 1)
    s = 1
    while s < x.shape[1]:
        rolled = pltpu.roll(x, s, 1)
        x = x + jnp.where(col >= s, rolled, 0.0)
        s *= 2
    return x


def _importance_topk_kernel(scores_ref, amask_ref, idx_ref, pmask_ref, acc_ref):
    h = pl.program_id(0)
    qi = pl.program_id(1)
    step = h * NQ + qi

    @pl.when(step == 0)
    def _():
        acc_ref[...] = jnp.zeros_like(acc_ref)

    blk = scores_ref[0]
    acc_ref[...] += jnp.sum(blk, axis=0, keepdims=True)

    @pl.when(step == H * NQ - 1)
    def _():
        x = acc_ref[...]
        bits = lax.bitcast_convert_type(x, jnp.int32)
        skey = jnp.where(bits >= 0, bits, bits ^ jnp.int32(0x7FFFFFFF))

        def bit_step(i, p):
            trial = p + (jnp.int32(1) << (jnp.int32(31) - i))
            c = jnp.sum((skey >= trial).astype(jnp.int32))
            return jnp.where(c >= K, trial, p)

        t = lax.fori_loop(0, 32, bit_step, jnp.int32(-2147483648))

        gt = skey > t
        eq = skey == t
        need = K - jnp.sum(gt.astype(jnp.int32))
        cum_eq = _lane_cumsum(eq.astype(jnp.float32))
        sel = gt | (eq & (cum_eq <= need.astype(jnp.float32)))

        rank = (_lane_cumsum(sel.astype(jnp.float32)) - 1.0).astype(jnp.int32)
        j = lax.broadcasted_iota(jnp.int32, (K + 1, L), 0)
        onehot = jnp.where((rank == j - 1) & sel, 1.0, 0.0)
        lpos = lax.broadcasted_iota(jnp.int32, (K + 1, L), 1).astype(
            jnp.float32)
        idx_ref[...] = jnp.sum(onehot * lpos, axis=1, keepdims=True).astype(
            jnp.int32).reshape(1, K + 1)
        pmask_ref[...] = jnp.sum(onehot * amask_ref[...], axis=1,
                                 keepdims=True).reshape(1, K + 1)


_importance_topk = pl.pallas_call(
    _importance_topk_kernel,
    grid=GRID,
    in_specs=[
        pl.BlockSpec((1, BQ, L), lambda h, qi: (h, qi, 0)),
        pl.BlockSpec((1, L), lambda h, qi: (0, 0)),
    ],
    out_specs=[
        pl.BlockSpec((1, K + 1), lambda h, qi: (0, 0)),
        pl.BlockSpec((1, K + 1), lambda h, qi: (0, 0)),
    ],
    out_shape=[
        jax.ShapeDtypeStruct((1, K + 1), jnp.int32),
        jax.ShapeDtypeStruct((1, K + 1), jnp.float32),
    ],
    scratch_shapes=[pltpu.VMEM((1, L), jnp.float32)],
)


@functools.cache
def _make_sc_gather():
    info = plsc.get_sparse_core_info()
    nc, ns = info.num_cores, info.num_subcores
    nw = nc * ns
    rows_per_w = (K + 1) // nw

    @functools.partial(
        pl.kernel,
        out_type=jax.ShapeDtypeStruct((K + 1, D), jnp.float32),
        mesh=plsc.VectorSubcoreMesh(core_axis_name="c", subcore_axis_name="s"),
        scratch_types=[
            pltpu.VMEM((rows_per_w,), jnp.int32),
            pltpu.VMEM((rows_per_w, D), jnp.float32),
            pltpu.VMEM((1,), jnp.int32),
            pltpu.VMEM((1, D), jnp.float32),
            pltpu.SemaphoreType.DMA,
        ],
    )
    def _sc_gather(table_hbm, idx_hbm, out_hbm, idx_v, rows_v, idx_e, row_e,
                   sem):
        wid = lax.axis_index("s") * nc + lax.axis_index("c")
        base = wid * rows_per_w
        pltpu.sync_copy(idx_hbm.at[pl.ds(base, rows_per_w)], idx_v)
        pltpu.async_copy(table_hbm.at[idx_v], rows_v, sem).wait()
        pltpu.sync_copy(rows_v, out_hbm.at[pl.ds(base, rows_per_w)])

        @pl.when(wid == nw - 1)
        def _():
            last = nw * rows_per_w
            pltpu.sync_copy(idx_hbm.at[pl.ds(last, 1)], idx_e)
            pltpu.async_copy(table_hbm.at[idx_e], row_e, sem).wait()
            pltpu.sync_copy(row_e, out_hbm.at[pl.ds(last, 1)])

    return _sc_gather


def kernel(hidden_states, attention_mask, self_attention_scores,
           Wq, bq, Wk, bk, Wv, bv, Wo, bo):
    b = hidden_states.shape[0]
    scores = self_attention_scores.reshape(H, L, L)
    amask_row = attention_mask.reshape(1, L)

    idx_full, pmask_full = _importance_topk(scores, amask_row)

    table = hidden_states.reshape(L, D)
    final = _make_sc_gather()(table, idx_full.reshape(K + 1))

    final_token = final.reshape(b, K + 1, D)
    final_attention_mask = pmask_full.reshape(b, 1, 1, K + 1)
    return (final_token, final_attention_mask)


# speedup vs baseline: 2.4873x; 1.0255x over previous
import functools

import jax
import jax.numpy as jnp
from jax import lax
from jax.experimental import pallas as pl
from jax.experimental.pallas import tpu as pltpu
from jax.experimental.pallas import tpu_sc as plsc

L = 2048
D = 2048
H = 16
K = 512
BQ = 1024
NQ = L // BQ
GRID = (H, NQ)


def _lane_cumsum(x):
    col = lax.broadcasted_iota(jnp.int32, x.shape, 1)
    s = 1
    while s < x.shape[1]:
        rolled = pltpu.roll(x, s, 1)
        x = x + jnp.where(col >= s, rolled, 0.0)
        s *= 2
    return x


def _importance_topk_kernel(scores_ref, amask_ref, idx_ref, pmask_ref, acc_ref):
    h = pl.program_id(0)
    qi = pl.program_id(1)
    step = h * NQ + qi

    @pl.when(step == 0)
    def _():
        acc_ref[...] = jnp.zeros_like(acc_ref)

    blk = scores_ref[0]
    acc_ref[...] += jnp.sum(blk, axis=0, keepdims=True)

    @pl.when(step == H * NQ - 1)
    def _():
        x = acc_ref[...]
        bits = lax.bitcast_convert_type(x, jnp.int32)
        skey = jnp.where(bits >= 0, bits, bits ^ jnp.int32(0x7FFFFFFF))

        riota = lax.broadcasted_iota(jnp.int32, (16, 1), 0)
        t = jnp.int32(-2147483648)
        for shift in range(28, -1, -4):
            trials = t + (riota << shift)
            cmp = (skey >= trials).astype(jnp.float32)
            counts = jnp.sum(cmp, axis=1, keepdims=True)
            r_sel = jnp.sum((counts >= float(K)).astype(jnp.int32)) - 1
            t = t + (r_sel << shift)

        gt = skey > t
        eq = skey == t
        need = K - jnp.sum(gt.astype(jnp.int32))
        cum_eq = _lane_cumsum(eq.astype(jnp.float32))
        sel = gt | (eq & (cum_eq <= need.astype(jnp.float32)))

        rank = (_lane_cumsum(sel.astype(jnp.float32)) - 1.0).astype(jnp.int32)
        j = lax.broadcasted_iota(jnp.int32, (K + 1, L), 0)
        onehot = jnp.where((rank == j - 1) & sel, 1.0, 0.0)
        lpos = lax.broadcasted_iota(jnp.int32, (K + 1, L), 1).astype(
            jnp.float32)
        idx_ref[...] = jnp.sum(onehot * lpos, axis=1, keepdims=True).astype(
            jnp.int32).reshape(1, K + 1)
        pmask_ref[...] = jnp.sum(onehot * amask_ref[...], axis=1,
                                 keepdims=True).reshape(1, K + 1)


_importance_topk = pl.pallas_call(
    _importance_topk_kernel,
    grid=GRID,
    in_specs=[
        pl.BlockSpec((1, BQ, L), lambda h, qi: (h, qi, 0)),
        pl.BlockSpec((1, L), lambda h, qi: (0, 0)),
    ],
    out_specs=[
        pl.BlockSpec((1, K + 1), lambda h, qi: (0, 0)),
        pl.BlockSpec((1, K + 1), lambda h, qi: (0, 0)),
    ],
    out_shape=[
        jax.ShapeDtypeStruct((1, K + 1), jnp.int32),
        jax.ShapeDtypeStruct((1, K + 1), jnp.float32),
    ],
    scratch_shapes=[pltpu.VMEM((1, L), jnp.float32)],
)


@functools.cache
def _make_sc_gather():
    info = plsc.get_sparse_core_info()
    nc, ns = info.num_cores, info.num_subcores
    nw = nc * ns
    rows_per_w = (K + 1) // nw

    @functools.partial(
        pl.kernel,
        out_type=jax.ShapeDtypeStruct((K + 1, D), jnp.float32),
        mesh=plsc.VectorSubcoreMesh(core_axis_name="c", subcore_axis_name="s"),
        scratch_types=[
            pltpu.VMEM((rows_per_w,), jnp.int32),
            pltpu.VMEM((rows_per_w, D), jnp.float32),
            pltpu.VMEM((1,), jnp.int32),
            pltpu.VMEM((1, D), jnp.float32),
            pltpu.SemaphoreType.DMA,
        ],
    )
    def _sc_gather(table_hbm, idx_hbm, out_hbm, idx_v, rows_v, idx_e, row_e,
                   sem):
        wid = lax.axis_index("s") * nc + lax.axis_index("c")
        base = wid * rows_per_w
        pltpu.sync_copy(idx_hbm.at[pl.ds(base, rows_per_w)], idx_v)
        pltpu.async_copy(table_hbm.at[idx_v], rows_v, sem).wait()
        pltpu.sync_copy(rows_v, out_hbm.at[pl.ds(base, rows_per_w)])

        @pl.when(wid == nw - 1)
        def _():
            last = nw * rows_per_w
            pltpu.sync_copy(idx_hbm.at[pl.ds(last, 1)], idx_e)
            pltpu.async_copy(table_hbm.at[idx_e], row_e, sem).wait()
            pltpu.sync_copy(row_e, out_hbm.at[pl.ds(last, 1)])

    return _sc_gather


def kernel(hidden_states, attention_mask, self_attention_scores,
           Wq, bq, Wk, bk, Wv, bv, Wo, bo):
    b = hidden_states.shape[0]
    scores = self_attention_scores.reshape(H, L, L)
    amask_row = attention_mask.reshape(1, L)

    idx_full, pmask_full = _importance_topk(scores, amask_row)

    table = hidden_states.reshape(L, D)
    final = _make_sc_gather()(table, idx_full.reshape(K + 1))

    final_token = final.reshape(b, K + 1, D)
    final_attention_mask = pmask_full.reshape(b, 1, 1, K + 1)
    return (final_token, final_attention_mask)


# speedup vs baseline: 2.5203x; 1.0133x over previous
import functools

import jax
import jax.numpy as jnp
from jax import lax
from jax.experimental import pallas as pl
from jax.experimental.pallas import tpu as pltpu
from jax.experimental.pallas import tpu_sc as plsc

L = 2048
D = 2048
H = 16
K = 512
BQ = 1024
NQ = L // BQ
GRID = (H, NQ)


def _lane_cumsum(x):
    col = lax.broadcasted_iota(jnp.int32, x.shape, 1)
    s = 1
    while s < x.shape[1]:
        rolled = pltpu.roll(x, s, 1)
        x = x + jnp.where(col >= s, rolled, 0.0)
        s *= 2
    return x


def _importance_topk_kernel(scores_ref, amask_ref, idx_ref, pmask_ref, acc_ref):
    h = pl.program_id(0)
    qi = pl.program_id(1)
    step = h * NQ + qi

    @pl.when(step == 0)
    def _():
        acc_ref[...] = jnp.zeros_like(acc_ref)

    blk = scores_ref[0]
    acc_ref[...] += jnp.sum(blk, axis=0, keepdims=True)

    @pl.when(step == H * NQ - 1)
    def _():
        x = acc_ref[...]
        bits = lax.bitcast_convert_type(x, jnp.int32)
        skey = jnp.where(bits >= 0, bits, bits ^ jnp.int32(0x7FFFFFFF))

        riota = lax.broadcasted_iota(jnp.int32, (16, 1), 0)
        t = jnp.int32(-2147483648)
        for shift in range(28, -1, -4):
            trials = t + (riota << shift)
            cmp = (skey >= trials).astype(jnp.float32)
            counts = jnp.sum(cmp, axis=1, keepdims=True)
            r_sel = jnp.sum((counts >= float(K)).astype(jnp.int32)) - 1
            t = t + (r_sel << shift)

        gt = skey > t
        eq = skey == t
        need = K - jnp.sum(gt.astype(jnp.int32))
        cum_eq = _lane_cumsum(eq.astype(jnp.float32))
        sel = gt | (eq & (cum_eq <= need.astype(jnp.float32)))

        rank = (_lane_cumsum(sel.astype(jnp.float32)) - 1.0).astype(jnp.int32)
        j = lax.broadcasted_iota(jnp.int32, (K + 1, L), 0)
        onehot = jnp.where((rank == j - 1) & sel, 1.0, 0.0)
        lpos = lax.broadcasted_iota(jnp.int32, (K + 1, L), 1).astype(
            jnp.float32)
        idx_ref[...] = jnp.sum(onehot * lpos, axis=1, keepdims=True).astype(
            jnp.int32).reshape(1, K + 1)
        pmask_ref[...] = jnp.sum(onehot * amask_ref[...], axis=1,
                                 keepdims=True).reshape(1, K + 1)


_importance_topk = pl.pallas_call(
    _importance_topk_kernel,
    grid=GRID,
    in_specs=[
        pl.BlockSpec((1, BQ, L), lambda h, qi: (h, qi, 0)),
        pl.BlockSpec((1, L), lambda h, qi: (0, 0)),
    ],
    out_specs=[
        pl.BlockSpec((1, K + 1), lambda h, qi: (0, 0)),
        pl.BlockSpec((1, K + 1), lambda h, qi: (0, 0)),
    ],
    out_shape=[
        jax.ShapeDtypeStruct((1, K + 1), jnp.int32),
        jax.ShapeDtypeStruct((1, K + 1), jnp.float32),
    ],
    scratch_shapes=[pltpu.VMEM((1, L), jnp.float32)],
)


@functools.cache
def _make_sc_gather():
    info = plsc.get_sparse_core_info()
    nc, ns = info.num_cores, info.num_subcores
    nw = nc * ns
    rows_per_w = (K + 1) // nw

    @functools.partial(
        pl.kernel,
        out_type=jax.ShapeDtypeStruct((K + 1, D), jnp.float32),
        mesh=plsc.VectorSubcoreMesh(core_axis_name="c", subcore_axis_name="s"),
        scratch_types=[
            pltpu.VMEM((rows_per_w,), jnp.int32),
            pltpu.VMEM((rows_per_w, D), jnp.float32),
            pltpu.VMEM((1,), jnp.int32),
            pltpu.VMEM((1, D), jnp.float32),
            pltpu.SemaphoreType.DMA,
            pltpu.SemaphoreType.DMA,
            pltpu.SemaphoreType.DMA,
            pltpu.SemaphoreType.DMA,
        ],
    )
    def _sc_gather(table_hbm, idx_hbm, out_hbm, idx_v, rows_v, idx_e, row_e,
                   sem0, sem1, wsem0, wsem1):
        half = rows_per_w // 2
        wid = lax.axis_index("s") * nc + lax.axis_index("c")
        base = wid * rows_per_w
        pltpu.sync_copy(idx_hbm.at[pl.ds(base, rows_per_w)], idx_v)
        g0 = pltpu.async_copy(table_hbm.at[idx_v.at[pl.ds(0, half)]],
                              rows_v.at[pl.ds(0, half)], sem0)
        g1 = pltpu.async_copy(table_hbm.at[idx_v.at[pl.ds(half, half)]],
                              rows_v.at[pl.ds(half, half)], sem1)
        g0.wait()
        w0 = pltpu.async_copy(rows_v.at[pl.ds(0, half)],
                              out_hbm.at[pl.ds(base, half)], wsem0)
        g1.wait()
        w1 = pltpu.async_copy(rows_v.at[pl.ds(half, half)],
                              out_hbm.at[pl.ds(base + half, half)], wsem1)

        @pl.when(wid == nw - 1)
        def _():
            last = nw * rows_per_w
            pltpu.sync_copy(idx_hbm.at[pl.ds(last, 1)], idx_e)
            pltpu.async_copy(table_hbm.at[idx_e], row_e, sem0).wait()
            pltpu.sync_copy(row_e, out_hbm.at[pl.ds(last, 1)])

        w0.wait()
        w1.wait()

    return _sc_gather


def kernel(hidden_states, attention_mask, self_attention_scores,
           Wq, bq, Wk, bk, Wv, bv, Wo, bo):
    b = hidden_states.shape[0]
    scores = self_attention_scores.reshape(H, L, L)
    amask_row = attention_mask.reshape(1, L)

    idx_full, pmask_full = _importance_topk(scores, amask_row)

    table = hidden_states.reshape(L, D)
    final = _make_sc_gather()(table, idx_full.reshape(K + 1))

    final_token = final.reshape(b, K + 1, D)
    final_attention_mask = pmask_full.reshape(b, 1, 1, K + 1)
    return (final_token, final_attention_mask)
